# Initial kernel scaffold; baseline (speedup 1.0000x reference)
import jax
import jax.numpy as jnp
from jax.experimental import pallas as pl


def kernel(vertex, nh_indices, center_weight, nh_weight, bias):
    raise NotImplementedError("write your pallas kernel here")


# speedup vs baseline: 1.2960x; 1.2960x over previous
import functools

import jax
import jax.numpy as jnp
from jax import lax
from jax.experimental import pallas as pl
from jax.experimental.pallas import tpu as pltpu
from jax.experimental.pallas import tpu_sc as plsc

N = 10000
NH = 32
D = 128
LANES = 16
VPR = D // LANES

NC = 2
NS = 16
NW = NC * NS
NPAD = 10240
NPW = NPAD // NW


def _mm_body(x_ref, wc_ref, wn_ref, b_ref, zc_ref, zn_ref):
    x = x_ref[...]
    zc_ref[...] = (
        jnp.dot(x, wc_ref[...], preferred_element_type=jnp.float32)
        + b_ref[...]
    )
    zn_ref[...] = jnp.dot(
        x, wn_ref[...], preferred_element_type=jnp.float32
    ) * jnp.float32(1.0 / NH)


def _matmuls(xpad, wc, wn, bias):
    blk = 1024
    return pl.pallas_call(
        _mm_body,
        grid=(NPAD // blk,),
        in_specs=[
            pl.BlockSpec((blk, D), lambda i: (i, 0)),
            pl.BlockSpec((D, D), lambda i: (0, 0)),
            pl.BlockSpec((D, D), lambda i: (0, 0)),
            pl.BlockSpec((1, D), lambda i: (0, 0)),
        ],
        out_specs=[
            pl.BlockSpec((blk, D), lambda i: (i, 0)),
            pl.BlockSpec((blk, D), lambda i: (i, 0)),
        ],
        out_shape=[
            jax.ShapeDtypeStruct((NPAD, D), jnp.float32),
            jax.ShapeDtypeStruct((NPAD, D), jnp.float32),
        ],
    )(xpad, wc, wn, bias.reshape(1, D))


def _accum_node(rows, zc_v, out_v, n):
    def row(r, accs):
        return tuple(accs[v] + rows[r, pl.ds(LANES * v, LANES)]
                     for v in range(VPR))

    accs = lax.fori_loop(
        0, NH, row,
        tuple(zc_v[n, pl.ds(LANES * v, LANES)] for v in range(VPR)),
        unroll=4)
    for v in range(VPR):
        out_v[n, pl.ds(LANES * v, LANES)] = jnp.maximum(
            accs[v], jnp.float32(0.0))


def _agg_body(zn_hbm, zc_hbm, idx_hbm, out_hbm,
              idx_v, rows_v, zc_v, out_v, sem0, sem1):
    wid = lax.axis_index("s") * NC + lax.axis_index("c")
    base = wid * NPW

    pltpu.sync_copy(idx_hbm.at[pl.ds(base, NPW)], idx_v)
    pltpu.sync_copy(zc_hbm.at[pl.ds(base, NPW)], zc_v)

    def gather(n, buf, sem):
        return pltpu.async_copy(zn_hbm.at[idx_v.at[n]], rows_v.at[buf], sem)

    gather(0, 0, sem0)
    gather(1, 1, sem1)

    def pair(p, carry):
        n0 = 2 * p
        pltpu.make_async_copy(zn_hbm.at[idx_v.at[n0]], rows_v.at[0], sem0).wait()
        _accum_node(rows_v.at[0], zc_v, out_v, n0)
        gather(jnp.minimum(n0 + 2, NPW - 1), 0, sem0)
        pltpu.make_async_copy(zn_hbm.at[idx_v.at[n0 + 1]], rows_v.at[1], sem1).wait()
        _accum_node(rows_v.at[1], zc_v, out_v, n0 + 1)
        gather(jnp.minimum(n0 + 3, NPW - 1), 1, sem1)
        return carry

    lax.fori_loop(0, NPW // 2, pair, 0)

    pltpu.make_async_copy(zn_hbm.at[idx_v.at[0]], rows_v.at[0], sem0).wait()
    pltpu.make_async_copy(zn_hbm.at[idx_v.at[0]], rows_v.at[1], sem1).wait()

    pltpu.sync_copy(out_v, out_hbm.at[pl.ds(base, NPW)])


@functools.partial(
    pl.kernel,
    out_type=jax.ShapeDtypeStruct((NPAD, D), jnp.float32),
    mesh=plsc.VectorSubcoreMesh(core_axis_name="c", subcore_axis_name="s"),
    scratch_types=[
        pltpu.VMEM((NPW, NH), jnp.int32),
        pltpu.VMEM((2, NH, D), jnp.float32),
        pltpu.VMEM((NPW, D), jnp.float32),
        pltpu.VMEM((NPW, D), jnp.float32),
        pltpu.SemaphoreType.DMA,
        pltpu.SemaphoreType.DMA,
    ],
)
def _aggregate(zn_hbm, zc_hbm, idx_hbm, out_hbm,
               idx_v, rows_v, zc_v, out_v, sem0, sem1):
    _agg_body(zn_hbm, zc_hbm, idx_hbm, out_hbm,
              idx_v, rows_v, zc_v, out_v, sem0, sem1)


def kernel(vertex, nh_indices, center_weight, nh_weight, bias):
    xpad = jnp.zeros((NPAD, D), jnp.float32).at[:N].set(vertex)
    idx = jnp.zeros((NPAD, NH), jnp.int32).at[:N].set(
        nh_indices.astype(jnp.int32))
    zc, zn = _matmuls(xpad, center_weight, nh_weight, bias)
    out = _aggregate(zn, zc, idx)
    return out[:N]


# speedup vs baseline: 1.3286x; 1.0251x over previous
import functools

import jax
import jax.numpy as jnp
from jax import lax
from jax.experimental import pallas as pl
from jax.experimental.pallas import tpu as pltpu
from jax.experimental.pallas import tpu_sc as plsc

N = 10000
NH = 32
D = 128
LANES = 16
VPR = D // LANES

NC = 2
NS = 16
NW = NC * NS
NPAD = 10240
NPW = NPAD // NW

GROUP = 4
NGRP = NPW // GROUP
NBUF = 4
NCHUNK = 4
CGRP = NGRP // NCHUNK
WAVES = CGRP // NBUF


def _mm_body(x_ref, wc_ref, wn_ref, b_ref, zc_ref, zn_ref):
    x = x_ref[...]
    zc_ref[...] = (
        jnp.dot(x, wc_ref[...], preferred_element_type=jnp.float32)
        + b_ref[...]
    )
    zn_ref[...] = jnp.dot(
        x, wn_ref[...], preferred_element_type=jnp.float32
    ) * jnp.float32(1.0 / NH)


def _matmuls(xpad, wc, wn, bias):
    blk = 1024
    return pl.pallas_call(
        _mm_body,
        grid=(NPAD // blk,),
        in_specs=[
            pl.BlockSpec((blk, D), lambda i: (i, 0)),
            pl.BlockSpec((D, D), lambda i: (0, 0)),
            pl.BlockSpec((D, D), lambda i: (0, 0)),
            pl.BlockSpec((1, D), lambda i: (0, 0)),
        ],
        out_specs=[
            pl.BlockSpec((blk, D), lambda i: (i, 0)),
            pl.BlockSpec((blk, D), lambda i: (i, 0)),
        ],
        out_shape=[
            jax.ShapeDtypeStruct((NPAD, D), jnp.float32),
            jax.ShapeDtypeStruct((NPAD, D), jnp.float32),
        ],
    )(xpad, wc, wn, bias.reshape(1, D))


def _accum_node(rows, j, zc_v, out_v, ln):
    def row(r, accs):
        return tuple(accs[v] + rows[j * NH + r, pl.ds(LANES * v, LANES)]
                     for v in range(VPR))

    accs = lax.fori_loop(
        0, NH, row,
        tuple(zc_v[ln, pl.ds(LANES * v, LANES)] for v in range(VPR)),
        unroll=4)
    for v in range(VPR):
        out_v[ln, pl.ds(LANES * v, LANES)] = jnp.maximum(
            accs[v], jnp.float32(0.0))


def _agg_body(zn_hbm, zc_hbm, idx_hbm, out_hbm,
              idx_v, rows_v, zc_v, out_v, sems):
    wid = lax.axis_index("s") * NC + lax.axis_index("c")
    base_n = wid * NPW
    base_g = wid * NGRP

    pltpu.sync_copy(idx_hbm.at[pl.ds(base_g, NGRP)], idx_v)

    def gather(g, b):
        return pltpu.async_copy(
            zn_hbm.at[idx_v.at[g]], rows_v.at[b], sems.at[b])

    for b in range(NBUF):
        gather(b, b)

    for c in range(NCHUNK):
        pltpu.sync_copy(zc_hbm.at[pl.ds(base_n + c * CGRP * GROUP,
                                        CGRP * GROUP)], zc_v)

        def wave(w, carry):
            for b in range(NBUF):
                gl = w * NBUF + b
                g = c * CGRP + gl
                pltpu.make_async_copy(
                    zn_hbm.at[idx_v.at[g]], rows_v.at[b], sems.at[b]).wait()
                for j in range(GROUP):
                    _accum_node(rows_v.at[b], j, zc_v, out_v,
                                gl * GROUP + j)

                @pl.when(g + NBUF < NGRP)
                def _():
                    gather(g + NBUF, b)
            return carry

        lax.fori_loop(0, WAVES, wave, 0)

        pltpu.sync_copy(
            out_v, out_hbm.at[pl.ds(base_n + c * CGRP * GROUP,
                                    CGRP * GROUP)])


@functools.partial(
    pl.kernel,
    out_type=jax.ShapeDtypeStruct((NPAD, D), jnp.float32),
    mesh=plsc.VectorSubcoreMesh(core_axis_name="c", subcore_axis_name="s"),
    scratch_types=[
        pltpu.VMEM((NGRP, GROUP * NH), jnp.int32),
        pltpu.VMEM((NBUF, GROUP * NH, D), jnp.float32),
        pltpu.VMEM((CGRP * GROUP, D), jnp.float32),
        pltpu.VMEM((CGRP * GROUP, D), jnp.float32),
        pltpu.SemaphoreType.DMA((NBUF,)),
    ],
)
def _aggregate(zn_hbm, zc_hbm, idx_hbm, out_hbm,
               idx_v, rows_v, zc_v, out_v, sems):
    _agg_body(zn_hbm, zc_hbm, idx_hbm, out_hbm,
              idx_v, rows_v, zc_v, out_v, sems)


def kernel(vertex, nh_indices, center_weight, nh_weight, bias):
    xpad = jnp.zeros((NPAD, D), jnp.float32).at[:N].set(vertex)
    idx = jnp.zeros((NPAD, NH), jnp.int32).at[:N].set(
        nh_indices.astype(jnp.int32))
    idx_g = idx.reshape(NPAD // GROUP, GROUP * NH)
    zc, zn = _matmuls(xpad, center_weight, nh_weight, bias)
    out = _aggregate(zn, zc, idx_g)
    return out[:N]


# speedup vs baseline: 6.1936x; 4.6619x over previous
import functools

import jax
import jax.numpy as jnp
from jax import lax
from jax.experimental import pallas as pl
from jax.experimental.pallas import tpu as pltpu
from jax.experimental.pallas import tpu_sc as plsc

N = 10000
NH = 32
D = 128
LANES = 16
VPR = D // LANES

NC = 2
NS = 16
NW = NC * NS
NPAD = 10240
NPW = NPAD // NW

GROUP = 2
NGRP = NPW // GROUP
NBUF = 2
NCHUNK = 8
CGRP = NGRP // NCHUNK
WAVES = CGRP // NBUF


def _mm_body(x_ref, wc_ref, wn_ref, b_ref, zc_ref, zn_ref):
    x = x_ref[...]
    zc_ref[...] = (
        jnp.dot(x, wc_ref[...], preferred_element_type=jnp.float32)
        + b_ref[...]
    )
    zn_ref[...] = jnp.dot(
        x, wn_ref[...], preferred_element_type=jnp.float32
    ) * jnp.float32(1.0 / NH)


def _matmuls(xpad, wc, wn, bias):
    blk = 1024
    return pl.pallas_call(
        _mm_body,
        grid=(NPAD // blk,),
        in_specs=[
            pl.BlockSpec((blk, D), lambda i: (i, 0)),
            pl.BlockSpec((D, D), lambda i: (0, 0)),
            pl.BlockSpec((D, D), lambda i: (0, 0)),
            pl.BlockSpec((1, D), lambda i: (0, 0)),
        ],
        out_specs=[
            pl.BlockSpec((blk, D), lambda i: (i, 0)),
            pl.BlockSpec((blk, D), lambda i: (i, 0)),
        ],
        out_shape=[
            jax.ShapeDtypeStruct((NPAD, D), jnp.float32),
            jax.ShapeDtypeStruct((NPAD, D), jnp.float32),
        ],
    )(xpad, wc, wn, bias.reshape(1, D))


def _accum_node(rows, j, zc_v, out_v, ln):
    def row(r, accs):
        return tuple(accs[v] + rows[j * NH + r, pl.ds(LANES * v, LANES)]
                     for v in range(VPR))

    accs = lax.fori_loop(
        0, NH, row,
        tuple(zc_v[ln, pl.ds(LANES * v, LANES)] for v in range(VPR)),
        unroll=4)
    for v in range(VPR):
        out_v[ln, pl.ds(LANES * v, LANES)] = jnp.maximum(
            accs[v], jnp.float32(0.0))


def _agg_body(zn_hbm, zc_hbm, idx_hbm, out_hbm,
              idx_v, rows_v, zc_v, out_v, zn_sh, sems):
    sid = lax.axis_index("s")
    wid = sid * NC + lax.axis_index("c")
    base_n = wid * NPW
    base_g = wid * NGRP

    stage = NPAD // NS
    pltpu.sync_copy(zn_hbm.at[pl.ds(sid * stage, stage)],
                    zn_sh.at[pl.ds(sid * stage, stage)])

    pltpu.sync_copy(idx_hbm.at[pl.ds(base_g, NGRP)], idx_v)
    plsc.subcore_barrier()

    def gather(g, b):
        return pltpu.async_copy(
            zn_sh.at[idx_v.at[g]], rows_v.at[b], sems.at[b])

    for b in range(NBUF):
        gather(b, b)

    for c in range(NCHUNK):
        pltpu.sync_copy(zc_hbm.at[pl.ds(base_n + c * CGRP * GROUP,
                                        CGRP * GROUP)], zc_v)

        def wave(w, carry):
            for b in range(NBUF):
                gl = w * NBUF + b
                g = c * CGRP + gl
                pltpu.make_async_copy(
                    zn_hbm.at[idx_v.at[g]], rows_v.at[b], sems.at[b]).wait()
                for j in range(GROUP):
                    _accum_node(rows_v.at[b], j, zc_v, out_v,
                                gl * GROUP + j)

                @pl.when(g + NBUF < NGRP)
                def _():
                    gather(g + NBUF, b)
            return carry

        lax.fori_loop(0, WAVES, wave, 0)

        pltpu.sync_copy(
            out_v, out_hbm.at[pl.ds(base_n + c * CGRP * GROUP,
                                    CGRP * GROUP)])


@functools.partial(
    pl.kernel,
    out_type=jax.ShapeDtypeStruct((NPAD, D), jnp.float32),
    mesh=plsc.VectorSubcoreMesh(core_axis_name="c", subcore_axis_name="s"),
    scratch_types=[
        pltpu.VMEM((NGRP, GROUP * NH), jnp.int32),
        pltpu.VMEM((NBUF, GROUP * NH, D), jnp.float32),
        pltpu.VMEM((CGRP * GROUP, D), jnp.float32),
        pltpu.VMEM((CGRP * GROUP, D), jnp.float32),
        pltpu.VMEM_SHARED((NPAD, D), jnp.float32),
        pltpu.SemaphoreType.DMA((NBUF,)),
    ],
)
def _aggregate(zn_hbm, zc_hbm, idx_hbm, out_hbm,
               idx_v, rows_v, zc_v, out_v, zn_sh, sems):
    _agg_body(zn_hbm, zc_hbm, idx_hbm, out_hbm,
              idx_v, rows_v, zc_v, out_v, zn_sh, sems)


def kernel(vertex, nh_indices, center_weight, nh_weight, bias):
    xpad = jnp.zeros((NPAD, D), jnp.float32).at[:N].set(vertex)
    idx = jnp.zeros((NPAD, NH), jnp.int32).at[:N].set(
        nh_indices.astype(jnp.int32))
    idx_g = idx.reshape(NPAD // GROUP, GROUP * NH)
    zc, zn = _matmuls(xpad, center_weight, nh_weight, bias)
    out = _aggregate(zn, zc, idx_g)
    return out[:N]
